# baseline (speedup 1.0000x reference)
import functools

import jax
import jax.numpy as jnp
from jax import lax
from jax.experimental import pallas as pl
from jax.experimental.pallas import tpu as pltpu
from jax.experimental.pallas import tpu_sc as plsc


def _router_body(x_ref, wg_ref, logits_ref, comb_ref, xpack_ref):
    x = x_ref[...]
    wg = wg_ref[...]
    logits = jax.lax.dot_general(
        x, wg, (((1,), (1,)), ((), ())),
        preferred_element_type=jnp.float32,
    )
    logits_ref[...] = logits
    T, E = logits.shape
    m = jnp.max(logits, axis=1, keepdims=True)
    ex = jnp.exp(logits - m)
    p = ex / jnp.sum(ex, axis=1, keepdims=True)
    eidx = jax.lax.broadcasted_iota(jnp.int32, (T, E), 1)
    a1 = jnp.argmax(p, axis=1).astype(jnp.int32)
    m1 = jnp.max(p, axis=1)
    mask1 = eidx == a1[:, None]
    p2 = jnp.where(mask1, -1.0, p)
    a2 = jnp.argmax(p2, axis=1).astype(jnp.int32)
    m2 = jnp.max(p2, axis=1)
    denom = m1 + m2
    w1 = m1 / denom
    w2 = m2 / denom
    mask2 = eidx == a2[:, None]
    comb_ref[...] = (jnp.where(mask1, w1[:, None], 0.0)
                     + jnp.where(mask2, w2[:, None], 0.0))
    xpack_ref[...] = x.reshape(xpack_ref.shape)


def _sc_scatter_body(x_hbm, idx_hbm, o_hbm, idx_v, rows_v, sem, *, chunks):
    nc = 2
    bpw = idx_v.shape[0]
    wid = lax.axis_index("subcore") * nc + lax.axis_index("core")
    t = x_hbm.shape[0]
    for c in range(chunks):
        base = wid * (bpw * chunks) + c * bpw
        src = lax.rem(base, t)
        pltpu.sync_copy(idx_hbm.at[pl.ds(base, bpw)], idx_v)
        pltpu.sync_copy(x_hbm.at[pl.ds(src, bpw)], rows_v)
        pltpu.async_copy(rows_v, o_hbm.at[idx_v], sem).wait()


def _sc_gather_body(ys_hbm, idx_hbm, o_hbm, idx_v, rows_v, sem, *, chunks):
    nc = 2
    bpw = idx_v.shape[0]
    wid = lax.axis_index("subcore") * nc + lax.axis_index("core")
    for c in range(chunks):
        base = wid * (bpw * chunks) + c * bpw
        pltpu.sync_copy(idx_hbm.at[pl.ds(base, bpw)], idx_v)
        pltpu.async_copy(ys_hbm.at[idx_v], rows_v, sem).wait()
        pltpu.sync_copy(rows_v, o_hbm.at[pl.ds(base, bpw)])


_SC_MESH = dict(core_axis_name="core", subcore_axis_name="subcore")
_NW = 32


_CHUNKS = 2


def _sc_scatter_rows(x_pack, dest, p):
    a = dest.shape[0]
    _, sl, lanes = x_pack.shape
    bpw = a // (_NW * _CHUNKS)
    run = pl.kernel(
        functools.partial(_sc_scatter_body, chunks=_CHUNKS),
        out_type=jax.ShapeDtypeStruct((p, sl, lanes), x_pack.dtype),
        mesh=plsc.VectorSubcoreMesh(**_SC_MESH),
        scratch_types=[
            pltpu.VMEM((bpw,), jnp.int32),
            pltpu.VMEM((bpw, sl, lanes), x_pack.dtype),
            pltpu.SemaphoreType.DMA,
        ],
    )
    return run(x_pack, dest)


def _sc_gather_rows(ys, dest):
    a = dest.shape[0]
    _, sl, lanes = ys.shape
    bpw = a // (_NW * _CHUNKS)
    run = pl.kernel(
        functools.partial(_sc_gather_body, chunks=_CHUNKS),
        out_type=jax.ShapeDtypeStruct((a, sl, lanes), ys.dtype),
        mesh=plsc.VectorSubcoreMesh(**_SC_MESH),
        scratch_types=[
            pltpu.VMEM((bpw,), jnp.int32),
            pltpu.VMEM((bpw, sl, lanes), ys.dtype),
            pltpu.SemaphoreType.DMA,
        ],
    )
    return run(ys, dest)


def _gemm_body(be_ref, xs_ref, wg_ref, wu_ref, wd_ref, ys_ref):
    nb = pl.num_programs(0)

    @pl.when(pl.program_id(0) < be_ref[nb])
    def _compute():
        blk = xs_ref.shape[0]
        xb = xs_ref[...].reshape(blk, -1).astype(jnp.bfloat16)
        wg = wg_ref[0].astype(jnp.bfloat16)
        wu = wu_ref[0].astype(jnp.bfloat16)
        wd = wd_ref[0].astype(jnp.bfloat16)
        g = jax.lax.dot_general(xb, wg, (((1,), (1,)), ((), ())),
                                preferred_element_type=jnp.float32)
        u = jax.lax.dot_general(xb, wu, (((1,), (1,)), ((), ())),
                                preferred_element_type=jnp.float32)
        h = (g * jax.lax.logistic(g) * u).astype(jnp.bfloat16)
        y = jax.lax.dot_general(h, wd, (((1,), (1,)), ((), ())),
                                preferred_element_type=jnp.float32)
        ys_ref[...] = y.reshape(ys_ref.shape)


def _combine_body(y1_ref, y2_ref, w1_ref, w2_ref, out_ref):
    bt = y1_ref.shape[1]
    y1 = y1_ref[0].reshape(bt, -1).astype(jnp.float32)
    y2 = y2_ref[0].reshape(bt, -1).astype(jnp.float32)
    out_ref[...] = y1 * w1_ref[...] + y2 * w2_ref[...]


def kernel(hidden_states, W_gate, W_g, W_u, W_d):
    b, s, d = hidden_states.shape
    x = hidden_states.reshape(-1, d)
    T, D = x.shape
    E, F, _ = W_g.shape
    K = 2
    BLK = 128
    NB = T * K // BLK + E - 1
    P = NB * BLK
    SL = D // 128
    A = T * K
    NW = 32
    BPW = A // NW

    logits, comb, x_pack = pl.pallas_call(
        _router_body,
        out_shape=(
            jax.ShapeDtypeStruct((T, E), jnp.float32),
            jax.ShapeDtypeStruct((T, E), jnp.float32),
            jax.ShapeDtypeStruct((T, SL, 128), jnp.float32),
        ),
    )(x, W_gate)

    eidx = jnp.arange(E, dtype=jnp.int32)[None, :]
    w1 = jnp.max(comb, axis=1)
    e1 = jnp.argmax(comb, axis=1).astype(jnp.int32)
    comb2 = jnp.where(eidx == e1[:, None], -1.0, comb)
    w2 = jnp.max(comb2, axis=1)
    e2 = jnp.argmax(comb2, axis=1).astype(jnp.int32)

    flat_e = jnp.concatenate([e1, e2])
    oh = (flat_e[:, None] == eidx).astype(jnp.int32)
    csum = jnp.cumsum(oh, axis=0)
    rank = jnp.take_along_axis(csum, flat_e[:, None], axis=1)[:, 0] - 1
    counts = csum[-1]
    nblk = (counts + BLK - 1) // BLK
    seg_start_blk = jnp.concatenate(
        [jnp.zeros((1,), jnp.int32), jnp.cumsum(nblk)[:-1].astype(jnp.int32)])
    dest = seg_start_blk[flat_e] * BLK + rank
    seg_end_blk = seg_start_blk + nblk
    bidx = jnp.arange(NB, dtype=jnp.int32)
    blk_expert = jnp.minimum(
        jnp.sum((bidx[:, None] >= seg_end_blk[None, :]).astype(jnp.int32),
                axis=1), E - 1).astype(jnp.int32)

    xs = _sc_scatter_rows(x_pack, dest, P)

    be_used = jnp.concatenate([blk_expert, seg_end_blk[-1:]])
    grid_spec = pltpu.PrefetchScalarGridSpec(
        num_scalar_prefetch=1,
        grid=(NB,),
        in_specs=[
            pl.BlockSpec((BLK, SL, 128), lambda i, be: (i, 0, 0)),
            pl.BlockSpec((1, F, D), lambda i, be: (be[i], 0, 0)),
            pl.BlockSpec((1, F, D), lambda i, be: (be[i], 0, 0)),
            pl.BlockSpec((1, D, F), lambda i, be: (be[i], 0, 0)),
        ],
        out_specs=pl.BlockSpec((BLK, SL, 128), lambda i, be: (i, 0, 0)),
    )
    ys = pl.pallas_call(
        _gemm_body,
        grid_spec=grid_spec,
        out_shape=jax.ShapeDtypeStruct((P, SL, 128), jnp.float32),
    )(be_used, xs, W_g, W_u, W_d)

    y12 = _sc_gather_rows(ys, dest)
    y12 = y12.reshape(K, T, SL, 128)

    BT = 256
    out = pl.pallas_call(
        _combine_body,
        grid=(T // BT,),
        in_specs=[
            pl.BlockSpec((1, BT, SL, 128), lambda t: (0, t, 0, 0)),
            pl.BlockSpec((1, BT, SL, 128), lambda t: (1, t, 0, 0)),
            pl.BlockSpec((BT, 1), lambda t: (t, 0)),
            pl.BlockSpec((BT, 1), lambda t: (t, 0)),
        ],
        out_specs=pl.BlockSpec((BT, D), lambda t: (t, 0)),
        out_shape=jax.ShapeDtypeStruct((T, D), jnp.float32),
    )(y12, y12, w1[:, None], w2[:, None])

    return out.reshape(b, s, d), logits.reshape(b, s, E)


# speedup vs baseline: 1.3773x; 1.3773x over previous
import functools

import jax
import jax.numpy as jnp
from jax import lax
from jax.experimental import pallas as pl
from jax.experimental.pallas import tpu as pltpu
from jax.experimental.pallas import tpu_sc as plsc


def _lane_shift(x, k):
    e = x.shape[1]
    return jnp.concatenate(
        [jnp.zeros((1, k), x.dtype), x[:, :e - k]], axis=1)


def _router_body(x_ref, wg_ref, logits_ref, xpack_ref, dest_ref, be_ref,
                 w1_ref, w2_ref, *, blk, nb):
    x = x_ref[...]
    wg = wg_ref[...]
    logits = jax.lax.dot_general(
        x, wg, (((1,), (1,)), ((), ())),
        preferred_element_type=jnp.float32,
    )
    logits_ref[...] = logits
    T, E = logits.shape
    m = jnp.max(logits, axis=1, keepdims=True)
    ex = jnp.exp(logits - m)
    p = ex / jnp.sum(ex, axis=1, keepdims=True)
    eidx = jax.lax.broadcasted_iota(jnp.int32, (T, E), 1)
    a1 = jnp.argmax(p, axis=1).astype(jnp.int32)
    m1 = jnp.max(p, axis=1)
    mask1 = eidx == a1[:, None]
    p2 = jnp.where(mask1, -1.0, p)
    a2 = jnp.argmax(p2, axis=1).astype(jnp.int32)
    m2 = jnp.max(p2, axis=1)
    denom = m1 + m2
    w1_ref[...] = m1 / denom
    w2_ref[...] = m2 / denom
    mask2 = eidx == a2[:, None]
    xpack_ref[...] = x.reshape(xpack_ref.shape)

    C = T // 2
    ri = jax.lax.broadcasted_iota(jnp.int32, (C, C), 0)
    ci = jax.lax.broadcasted_iota(jnp.int32, (C, C), 1)
    tril = (ri >= ci).astype(jnp.bfloat16)
    chunks = [mask1[:C], mask1[C:], mask2[:C], mask2[C:]]
    inc = []
    prefix = jnp.zeros((1, E), jnp.float32)
    prefixes = []
    for mk in chunks:
        mb = mk.astype(jnp.bfloat16)
        cs = jax.lax.dot_general(tril, mb, (((1,), (0,)), ((), ())),
                                 preferred_element_type=jnp.float32)
        inc.append(cs)
        prefixes.append(prefix)
        prefix = prefix + cs[C - 1:C, :]
    counts = prefix
    rank_parts = []
    for mk, cs, pf in zip(chunks, inc, prefixes):
        mf = mk.astype(jnp.float32)
        rank_parts.append(jnp.sum((cs + pf) * mf, axis=1) - 1.0)
    counts_i = counts.astype(jnp.int32)
    nblk = (counts_i + blk - 1) // blk
    seg_end = nblk + _lane_shift(nblk, 1)
    seg_end = seg_end + _lane_shift(seg_end, 2)
    seg_end = seg_end + _lane_shift(seg_end, 4)
    seg_start = seg_end - nblk
    seg_start_f = seg_start.astype(jnp.float32)
    dest_parts = []
    for mk, rk in zip(chunks, rank_parts):
        sel = jnp.sum(seg_start_f * mk.astype(jnp.float32), axis=1)
        dest_parts.append((sel * blk + rk).astype(jnp.int32))
    dest_ref[...] = jnp.concatenate(dest_parts)
    nlanes = be_ref.shape[1]
    bi = jax.lax.broadcasted_iota(jnp.int32, (nlanes, E), 0)
    ge = (bi >= jnp.broadcast_to(seg_end, (nlanes, E))).astype(jnp.int32)
    bexp = jnp.minimum(jnp.sum(ge, axis=1), E - 1)
    lane = jax.lax.broadcasted_iota(jnp.int32, (nlanes,), 0)
    used = jnp.sum(nblk)
    be_ref[...] = jnp.where(lane < nb, bexp,
                            jnp.where(lane == nb, used, 0))[None, :]


def _sc_scatter_body(x_hbm, idx_hbm, o_hbm, idx_v, rows_v, sem, *, chunks):
    nc = 2
    bpw = idx_v.shape[0]
    wid = lax.axis_index("subcore") * nc + lax.axis_index("core")
    t = x_hbm.shape[0]
    for c in range(chunks):
        base = wid * (bpw * chunks) + c * bpw
        src = lax.rem(base, t)
        pltpu.sync_copy(idx_hbm.at[pl.ds(base, bpw)], idx_v)
        pltpu.sync_copy(x_hbm.at[pl.ds(src, bpw)], rows_v)
        pltpu.async_copy(rows_v, o_hbm.at[idx_v], sem).wait()


def _sc_gather_body(ys_hbm, idx_hbm, o_hbm, idx_v, rows_v, sem, *, chunks):
    nc = 2
    bpw = idx_v.shape[0]
    wid = lax.axis_index("subcore") * nc + lax.axis_index("core")
    for c in range(chunks):
        base = wid * (bpw * chunks) + c * bpw
        pltpu.sync_copy(idx_hbm.at[pl.ds(base, bpw)], idx_v)
        pltpu.async_copy(ys_hbm.at[idx_v], rows_v, sem).wait()
        pltpu.sync_copy(rows_v, o_hbm.at[pl.ds(base, bpw)])


_SC_MESH = dict(core_axis_name="core", subcore_axis_name="subcore")
_NW = 32


_CHUNKS = 2


def _sc_scatter_rows(x_pack, dest, p):
    a = dest.shape[0]
    _, sl, lanes = x_pack.shape
    bpw = a // (_NW * _CHUNKS)
    run = pl.kernel(
        functools.partial(_sc_scatter_body, chunks=_CHUNKS),
        out_type=jax.ShapeDtypeStruct((p, sl, lanes), x_pack.dtype),
        mesh=plsc.VectorSubcoreMesh(**_SC_MESH),
        scratch_types=[
            pltpu.VMEM((bpw,), jnp.int32),
            pltpu.VMEM((bpw, sl, lanes), x_pack.dtype),
            pltpu.SemaphoreType.DMA,
        ],
    )
    return run(x_pack, dest)


def _sc_gather_rows(ys, dest):
    a = dest.shape[0]
    _, sl, lanes = ys.shape
    bpw = a // (_NW * _CHUNKS)
    run = pl.kernel(
        functools.partial(_sc_gather_body, chunks=_CHUNKS),
        out_type=jax.ShapeDtypeStruct((a, sl, lanes), ys.dtype),
        mesh=plsc.VectorSubcoreMesh(**_SC_MESH),
        scratch_types=[
            pltpu.VMEM((bpw,), jnp.int32),
            pltpu.VMEM((bpw, sl, lanes), ys.dtype),
            pltpu.SemaphoreType.DMA,
        ],
    )
    return run(ys, dest)


def _gemm_body(be_ref, xs_ref, wg_ref, wu_ref, wd_ref, ys_ref):
    nb = pl.num_programs(0)

    @pl.when(pl.program_id(0) < be_ref[nb])
    def _compute():
        blk = xs_ref.shape[0]
        xb = xs_ref[...].reshape(blk, -1).astype(jnp.bfloat16)
        wg = wg_ref[0].astype(jnp.bfloat16)
        wu = wu_ref[0].astype(jnp.bfloat16)
        wd = wd_ref[0].astype(jnp.bfloat16)
        g = jax.lax.dot_general(xb, wg, (((1,), (1,)), ((), ())),
                                preferred_element_type=jnp.float32)
        u = jax.lax.dot_general(xb, wu, (((1,), (1,)), ((), ())),
                                preferred_element_type=jnp.float32)
        h = (g * jax.lax.logistic(g) * u).astype(jnp.bfloat16)
        y = jax.lax.dot_general(h, wd, (((1,), (1,)), ((), ())),
                                preferred_element_type=jnp.float32)
        ys_ref[...] = y.reshape(ys_ref.shape)


def _combine_body(y1_ref, y2_ref, w1_ref, w2_ref, out_ref):
    bt = y1_ref.shape[1]
    y1 = y1_ref[0].reshape(bt, -1).astype(jnp.float32)
    y2 = y2_ref[0].reshape(bt, -1).astype(jnp.float32)
    out_ref[...] = y1 * w1_ref[...] + y2 * w2_ref[...]


def kernel(hidden_states, W_gate, W_g, W_u, W_d):
    b, s, d = hidden_states.shape
    x = hidden_states.reshape(-1, d)
    T, D = x.shape
    E, F, _ = W_g.shape
    K = 2
    BLK = 256
    NB = T * K // BLK + E - 1
    P = NB * BLK
    SL = D // 128
    A = T * K
    NW = 32
    BPW = A // NW

    logits, x_pack, dest, be_out, w1, w2 = pl.pallas_call(
        functools.partial(_router_body, blk=BLK, nb=NB),
        out_shape=(
            jax.ShapeDtypeStruct((T, E), jnp.float32),
            jax.ShapeDtypeStruct((T, SL, 128), jnp.float32),
            jax.ShapeDtypeStruct((A,), jnp.int32),
            jax.ShapeDtypeStruct((1, 128), jnp.int32),
            jax.ShapeDtypeStruct((T,), jnp.float32),
            jax.ShapeDtypeStruct((T,), jnp.float32),
        ),
    )(x, W_gate)

    xs = _sc_scatter_rows(x_pack, dest, P)

    be_used = be_out[0, :NB + 1]
    grid_spec = pltpu.PrefetchScalarGridSpec(
        num_scalar_prefetch=1,
        grid=(NB,),
        in_specs=[
            pl.BlockSpec((BLK, SL, 128), lambda i, be: (i, 0, 0)),
            pl.BlockSpec((1, F, D), lambda i, be: (be[i], 0, 0)),
            pl.BlockSpec((1, F, D), lambda i, be: (be[i], 0, 0)),
            pl.BlockSpec((1, D, F), lambda i, be: (be[i], 0, 0)),
        ],
        out_specs=pl.BlockSpec((BLK, SL, 128), lambda i, be: (i, 0, 0)),
    )
    ys = pl.pallas_call(
        _gemm_body,
        grid_spec=grid_spec,
        out_shape=jax.ShapeDtypeStruct((P, SL, 128), jnp.float32),
    )(be_used, xs, W_g, W_u, W_d)

    y12 = _sc_gather_rows(ys, dest)
    y12 = y12.reshape(K, T, SL, 128)

    BT = 256
    out = pl.pallas_call(
        _combine_body,
        grid=(T // BT,),
        in_specs=[
            pl.BlockSpec((1, BT, SL, 128), lambda t: (0, t, 0, 0)),
            pl.BlockSpec((1, BT, SL, 128), lambda t: (1, t, 0, 0)),
            pl.BlockSpec((BT, 1), lambda t: (t, 0)),
            pl.BlockSpec((BT, 1), lambda t: (t, 0)),
        ],
        out_specs=pl.BlockSpec((BT, D), lambda t: (t, 0)),
        out_shape=jax.ShapeDtypeStruct((T, D), jnp.float32),
    )(y12, y12, w1[:, None], w2[:, None])

    return out.reshape(b, s, d), logits.reshape(b, s, E)


# speedup vs baseline: 1.3825x; 1.0038x over previous
import functools

import jax
import jax.numpy as jnp
from jax import lax
from jax.experimental import pallas as pl
from jax.experimental.pallas import tpu as pltpu
from jax.experimental.pallas import tpu_sc as plsc


def _lane_shift(x, k):
    e = x.shape[1]
    return jnp.concatenate(
        [jnp.zeros((1, k), x.dtype), x[:, :e - k]], axis=1)


def _router_body(x_ref, wg_ref, tril_ref, logits_ref, xpack_ref, dest_ref,
                 be_ref, w1_ref, w2_ref, *, blk, nb):
    x = x_ref[...]
    wg = wg_ref[...]
    logits = jax.lax.dot_general(
        x, wg, (((1,), (1,)), ((), ())),
        preferred_element_type=jnp.float32,
    )
    logits_ref[...] = logits
    T, E = logits.shape
    m = jnp.max(logits, axis=1, keepdims=True)
    ex = jnp.exp(logits - m)
    p = ex / jnp.sum(ex, axis=1, keepdims=True)
    eidx = jax.lax.broadcasted_iota(jnp.int32, (T, E), 1)
    a1 = jnp.argmax(p, axis=1).astype(jnp.int32)
    m1 = jnp.max(p, axis=1)
    mask1 = eidx == a1[:, None]
    p2 = jnp.where(mask1, -1.0, p)
    a2 = jnp.argmax(p2, axis=1).astype(jnp.int32)
    m2 = jnp.max(p2, axis=1)
    denom = m1 + m2
    w1_ref[...] = m1 / denom
    w2_ref[...] = m2 / denom
    mask2 = eidx == a2[:, None]
    xpack_ref[...] = x.reshape(xpack_ref.shape)

    C = T // 2
    tril = tril_ref[...]
    chunks = [mask1[:C], mask1[C:], mask2[:C], mask2[C:]]
    inc = []
    prefix = jnp.zeros((1, E), jnp.float32)
    prefixes = []
    for mk in chunks:
        mb = mk.astype(jnp.bfloat16)
        cs = jax.lax.dot_general(tril, mb, (((1,), (0,)), ((), ())),
                                 preferred_element_type=jnp.float32)
        inc.append(cs)
        prefixes.append(prefix)
        prefix = prefix + cs[C - 1:C, :]
    counts = prefix
    rank_parts = []
    for mk, cs, pf in zip(chunks, inc, prefixes):
        mf = mk.astype(jnp.float32)
        rank_parts.append(jnp.sum((cs + pf) * mf, axis=1) - 1.0)
    counts_i = counts.astype(jnp.int32)
    nblk = (counts_i + blk - 1) // blk
    seg_end = nblk + _lane_shift(nblk, 1)
    seg_end = seg_end + _lane_shift(seg_end, 2)
    seg_end = seg_end + _lane_shift(seg_end, 4)
    seg_start = seg_end - nblk
    seg_start_f = seg_start.astype(jnp.float32)
    dest_parts = []
    for mk, rk in zip(chunks, rank_parts):
        sel = jnp.sum(seg_start_f * mk.astype(jnp.float32), axis=1)
        dest_parts.append((sel * blk + rk).astype(jnp.int32))
    dest_ref[...] = jnp.concatenate(dest_parts)
    nlanes = be_ref.shape[1]
    bi = jax.lax.broadcasted_iota(jnp.int32, (nlanes, E), 0)
    ge = (bi >= jnp.broadcast_to(seg_end, (nlanes, E))).astype(jnp.int32)
    bexp = jnp.minimum(jnp.sum(ge, axis=1), E - 1)
    lane = jax.lax.broadcasted_iota(jnp.int32, (nlanes,), 0)
    used = jnp.sum(nblk)
    be_ref[...] = jnp.where(lane < nb, bexp,
                            jnp.where(lane == nb, used, 0))[None, :]


def _sc_scatter_body(x_hbm, idx_hbm, o_hbm, idx_v, rows_v, sem, *, chunks):
    nc = 2
    bpw = idx_v.shape[0]
    wid = lax.axis_index("subcore") * nc + lax.axis_index("core")
    t = x_hbm.shape[0]
    for c in range(chunks):
        base = wid * (bpw * chunks) + c * bpw
        src = lax.rem(base, t)
        pltpu.sync_copy(idx_hbm.at[pl.ds(base, bpw)], idx_v)
        pltpu.sync_copy(x_hbm.at[pl.ds(src, bpw)], rows_v)
        pltpu.async_copy(rows_v, o_hbm.at[idx_v], sem).wait()


def _sc_gather_body(ys_hbm, idx_hbm, o_hbm, idx_v, rows_v, sem, *, chunks):
    nc = 2
    bpw = idx_v.shape[0]
    wid = lax.axis_index("subcore") * nc + lax.axis_index("core")
    for c in range(chunks):
        base = wid * (bpw * chunks) + c * bpw
        pltpu.sync_copy(idx_hbm.at[pl.ds(base, bpw)], idx_v)
        pltpu.async_copy(ys_hbm.at[idx_v], rows_v, sem).wait()
        pltpu.sync_copy(rows_v, o_hbm.at[pl.ds(base, bpw)])


_SC_MESH = dict(core_axis_name="core", subcore_axis_name="subcore")
_NW = 32


_CHUNKS = 2


def _sc_scatter_rows(x_pack, dest, p):
    a = dest.shape[0]
    _, sl, lanes = x_pack.shape
    bpw = a // (_NW * _CHUNKS)
    run = pl.kernel(
        functools.partial(_sc_scatter_body, chunks=_CHUNKS),
        out_type=jax.ShapeDtypeStruct((p, sl, lanes), x_pack.dtype),
        mesh=plsc.VectorSubcoreMesh(**_SC_MESH),
        scratch_types=[
            pltpu.VMEM((bpw,), jnp.int32),
            pltpu.VMEM((bpw, sl, lanes), x_pack.dtype),
            pltpu.SemaphoreType.DMA,
        ],
    )
    return run(x_pack, dest)


def _sc_gather_rows(ys, dest):
    a = dest.shape[0]
    _, sl, lanes = ys.shape
    bpw = a // (_NW * _CHUNKS)
    run = pl.kernel(
        functools.partial(_sc_gather_body, chunks=_CHUNKS),
        out_type=jax.ShapeDtypeStruct((a, sl, lanes), ys.dtype),
        mesh=plsc.VectorSubcoreMesh(**_SC_MESH),
        scratch_types=[
            pltpu.VMEM((bpw,), jnp.int32),
            pltpu.VMEM((bpw, sl, lanes), ys.dtype),
            pltpu.SemaphoreType.DMA,
        ],
    )
    return run(ys, dest)


def _gemm_body(be_ref, xs_ref, wg_ref, wu_ref, wd_ref, ys_ref):
    nb = pl.num_programs(0)

    @pl.when(pl.program_id(0) < be_ref[nb])
    def _compute():
        blk = xs_ref.shape[0]
        xb = xs_ref[...].reshape(blk, -1).astype(jnp.bfloat16)
        wg = wg_ref[0].astype(jnp.bfloat16)
        wu = wu_ref[0].astype(jnp.bfloat16)
        wd = wd_ref[0].astype(jnp.bfloat16)
        g = jax.lax.dot_general(xb, wg, (((1,), (1,)), ((), ())),
                                preferred_element_type=jnp.float32)
        u = jax.lax.dot_general(xb, wu, (((1,), (1,)), ((), ())),
                                preferred_element_type=jnp.float32)
        h = (g * jax.lax.logistic(g) * u).astype(jnp.bfloat16)
        y = jax.lax.dot_general(h, wd, (((1,), (1,)), ((), ())),
                                preferred_element_type=jnp.float32)
        ys_ref[...] = y.reshape(ys_ref.shape)


def _combine_body(y1_ref, y2_ref, w1_ref, w2_ref, out_ref):
    bt = y1_ref.shape[1]
    y1 = y1_ref[0].reshape(bt, -1).astype(jnp.float32)
    y2 = y2_ref[0].reshape(bt, -1).astype(jnp.float32)
    out_ref[...] = y1 * w1_ref[...] + y2 * w2_ref[...]


def kernel(hidden_states, W_gate, W_g, W_u, W_d):
    b, s, d = hidden_states.shape
    x = hidden_states.reshape(-1, d)
    T, D = x.shape
    E, F, _ = W_g.shape
    K = 2
    BLK = 256
    NB = T * K // BLK + E - 1
    P = NB * BLK
    SL = D // 128
    A = T * K
    NW = 32
    BPW = A // NW

    tril_c = jnp.tril(jnp.ones((T // 2, T // 2), jnp.bfloat16))
    logits, x_pack, dest, be_out, w1, w2 = pl.pallas_call(
        functools.partial(_router_body, blk=BLK, nb=NB),
        out_shape=(
            jax.ShapeDtypeStruct((T, E), jnp.float32),
            jax.ShapeDtypeStruct((T, SL, 128), jnp.float32),
            jax.ShapeDtypeStruct((A,), jnp.int32),
            jax.ShapeDtypeStruct((1, 128), jnp.int32),
            jax.ShapeDtypeStruct((T,), jnp.float32),
            jax.ShapeDtypeStruct((T,), jnp.float32),
        ),
    )(x, W_gate, tril_c)

    xs = _sc_scatter_rows(x_pack, dest, P)

    be_used = be_out[0, :NB + 1]
    grid_spec = pltpu.PrefetchScalarGridSpec(
        num_scalar_prefetch=1,
        grid=(NB,),
        in_specs=[
            pl.BlockSpec((BLK, SL, 128), lambda i, be: (i, 0, 0)),
            pl.BlockSpec((1, F, D), lambda i, be: (be[i], 0, 0)),
            pl.BlockSpec((1, F, D), lambda i, be: (be[i], 0, 0)),
            pl.BlockSpec((1, D, F), lambda i, be: (be[i], 0, 0)),
        ],
        out_specs=pl.BlockSpec((BLK, SL, 128), lambda i, be: (i, 0, 0)),
    )
    ys = pl.pallas_call(
        _gemm_body,
        grid_spec=grid_spec,
        out_shape=jax.ShapeDtypeStruct((P, SL, 128), jnp.float32),
    )(be_used, xs, W_g, W_u, W_d)

    y12 = _sc_gather_rows(ys, dest)
    y12 = y12.reshape(K, T, SL, 128)

    BT = 512
    out = pl.pallas_call(
        _combine_body,
        grid=(T // BT,),
        in_specs=[
            pl.BlockSpec((1, BT, SL, 128), lambda t: (0, t, 0, 0)),
            pl.BlockSpec((1, BT, SL, 128), lambda t: (1, t, 0, 0)),
            pl.BlockSpec((BT, 1), lambda t: (t, 0)),
            pl.BlockSpec((BT, 1), lambda t: (t, 0)),
        ],
        out_specs=pl.BlockSpec((BT, D), lambda t: (t, 0)),
        out_shape=jax.ShapeDtypeStruct((T, D), jnp.float32),
    )(y12, y12, w1[:, None], w2[:, None])

    return out.reshape(b, s, d), logits.reshape(b, s, E)
